# baseline (speedup 1.0000x reference)
import functools

import jax
import jax.numpy as jnp
from jax import lax
from jax.experimental import pallas as pl
from jax.experimental.pallas import tpu as pltpu
from jax.experimental.pallas import tpu_sc as plsc

N_NODES = 10000
N_EDGES = 320000
IN_FEAT = 128
OUT2 = 128

D_AUG = 144
N_TAB = N_NODES + 16
ACC_ROWS = 10240
ROWS_PER_TILE = ACC_ROWS // 16
BATCH = 128

NC, NS = 2, 16
NW = NC * NS
BATCHES_PER_SUBCORE = 158
NB0, NB1 = 61, 97
E_PAD = NS * BATCHES_PER_SUBCORE * BATCH


def _sc_scatter(h_aug, src2d, dst2d, zinit):
    mesh = plsc.VectorSubcoreMesh(core_axis_name="c", subcore_axis_name="s")

    @functools.partial(
        pl.kernel,
        mesh=mesh,
        out_type=jax.ShapeDtypeStruct((NC, ACC_ROWS, D_AUG), jnp.float32),
        scratch_types=[
            pltpu.VMEM((BATCH,), jnp.int32),
            pltpu.VMEM((BATCH,), jnp.int32),
            pltpu.VMEM((2, BATCH), jnp.int32),
            pltpu.VMEM((2, BATCH, D_AUG), jnp.float32),
            pltpu.VMEM_SHARED((ACC_ROWS, D_AUG), jnp.float32),
            pltpu.SemaphoreType.DMA,
            [pltpu.SemaphoreType.DMA] * 2,
        ],
        compiler_params=pltpu.CompilerParams(use_tc_tiling_on_sc=False),
    )
    def k(tab_hbm, src_hbm, dst_hbm, z_hbm, out_hbm, idx_s0, idx_s1, idx_d,
          rows, acc, gsem, ssem):
        c = lax.axis_index("c")
        s = lax.axis_index("s")
        base = s * BATCHES_PER_SUBCORE + c * NB0
        nb = jnp.where(c == 0, NB0, NB1)
        idx_s = (idx_s0, idx_s1)

        pltpu.sync_copy(z_hbm, acc.at[pl.ds(s * ROWS_PER_TILE, ROWS_PER_TILE)])
        plsc.subcore_barrier()

        def load_gather(j, b):
            pltpu.sync_copy(src_hbm.at[base + j], idx_s[b])
            pltpu.sync_copy(dst_hbm.at[base + j], idx_d.at[b])
            pltpu.async_copy(tab_hbm.at[idx_s[b]], rows.at[b], gsem).wait()

        def scatter_start(b):
            pltpu.async_copy(rows.at[b], acc.at[idx_d.at[b]], ssem[b],
                             add=True)

        def scatter_wait(b):
            pltpu.make_async_copy(tab_hbm.at[pl.ds(0, BATCH)], rows.at[b],
                                  ssem[b]).wait()

        for b in range(2):
            load_gather(b, b)
            scatter_start(b)

        def body(g, carry):
            for b in range(2):
                j = 2 * g + b
                scatter_wait(b)
                load_gather(j, b)
                scatter_start(b)
            return carry

        lax.fori_loop(1, (nb - 1) // 2, body, 0)

        scatter_wait(0)
        load_gather(nb - 1, 0)
        scatter_start(0)
        scatter_wait(0)
        scatter_wait(1)
        plsc.subcore_barrier()

        pltpu.sync_copy(
            acc.at[pl.ds(s * ROWS_PER_TILE, ROWS_PER_TILE)],
            out_hbm.at[c, pl.ds(s * ROWS_PER_TILE, ROWS_PER_TILE)],
        )

    return k(h_aug, src2d, dst2d, zinit)


def _tc_finish_body(p0_ref, p1_ref, h_ref, w_ref, o_ref):
    p = p0_ref[...] + p1_ref[...]
    ssum = p[:, :IN_FEAT]
    deg = p[:, IN_FEAT:IN_FEAT + 1]
    r = jnp.maximum(deg, 1.0)
    agg = jnp.dot(ssum / r, w_ref[...], preferred_element_type=jnp.float32)
    o_ref[...] = jnp.maximum(agg, 0.0) + h_ref[...]


def _tc_finish(p0, p1, h, W):
    blk = 1000
    grid = (N_NODES // blk,)
    return pl.pallas_call(
        _tc_finish_body,
        grid=grid,
        in_specs=[
            pl.BlockSpec((blk, D_AUG), lambda i: (i, 0)),
            pl.BlockSpec((blk, D_AUG), lambda i: (i, 0)),
            pl.BlockSpec((blk, IN_FEAT), lambda i: (i, 0)),
            pl.BlockSpec((IN_FEAT, OUT2), lambda i: (0, 0)),
        ],
        out_specs=pl.BlockSpec((blk, OUT2), lambda i: (i, 0)),
        out_shape=jax.ShapeDtypeStruct((N_NODES, OUT2), jnp.float32),
    )(p0, p1, h, W)


@jax.jit
def kernel(h, edge_index, W):
    ei = edge_index.astype(jnp.int32)
    src = ei[0]
    dst = ei[1]
    pad = E_PAD - N_EDGES
    src_p = jnp.concatenate([src, jnp.full((pad,), N_NODES, jnp.int32)])
    dst_p = jnp.concatenate([dst, jnp.zeros((pad,), jnp.int32)])
    src2d = src_p.reshape(-1, BATCH)
    dst2d = dst_p.reshape(-1, BATCH)

    h_aug = jnp.zeros((N_TAB, D_AUG), jnp.float32)
    h_aug = h_aug.at[:N_NODES, :IN_FEAT].set(h)
    h_aug = h_aug.at[:N_NODES, IN_FEAT].set(1.0)

    zinit = jnp.zeros((ROWS_PER_TILE, D_AUG), jnp.float32)

    partials = _sc_scatter(h_aug, src2d, dst2d, zinit)
    return _tc_finish(partials[0], partials[1], h, W)


# speedup vs baseline: 1.1565x; 1.1565x over previous
import functools

import jax
import jax.numpy as jnp
from jax import lax
from jax.experimental import pallas as pl
from jax.experimental.pallas import tpu as pltpu
from jax.experimental.pallas import tpu_sc as plsc

N_NODES = 10000
N_EDGES = 320000
IN_FEAT = 128
OUT2 = 128

D_AUG = 144
N_TAB = N_NODES + 16
ACC_ROWS = 10240
ROWS_PER_TILE = ACC_ROWS // 16
BATCH = 128

NC, NS = 2, 16
NW = NC * NS
BATCHES_PER_SUBCORE = 158
NB0, NB1 = 97, 61
E_PAD = NS * BATCHES_PER_SUBCORE * BATCH


def _sc_scatter(h_aug, src2d, dst2d, zinit):
    mesh = plsc.VectorSubcoreMesh(core_axis_name="c", subcore_axis_name="s")

    @functools.partial(
        pl.kernel,
        mesh=mesh,
        out_type=jax.ShapeDtypeStruct((NC, ACC_ROWS, D_AUG), jnp.float32),
        scratch_types=[
            pltpu.VMEM((BATCH,), jnp.int32),
            pltpu.VMEM((BATCH,), jnp.int32),
            pltpu.VMEM((2, BATCH), jnp.int32),
            pltpu.VMEM((2, BATCH, D_AUG), jnp.float32),
            pltpu.VMEM_SHARED((ACC_ROWS, D_AUG), jnp.float32),
            pltpu.SemaphoreType.DMA,
            [pltpu.SemaphoreType.DMA] * 2,
        ],
        compiler_params=pltpu.CompilerParams(use_tc_tiling_on_sc=False),
    )
    def k(tab_hbm, src_hbm, dst_hbm, z_hbm, out_hbm, idx_s0, idx_s1, idx_d,
          rows, acc, gsem, ssem):
        c = lax.axis_index("c")
        s = lax.axis_index("s")
        base = s * BATCHES_PER_SUBCORE + c * NB0
        nb = jnp.where(c == 0, NB0, NB1)
        idx_s = (idx_s0, idx_s1)

        pltpu.sync_copy(z_hbm, acc.at[pl.ds(s * ROWS_PER_TILE, ROWS_PER_TILE)])
        plsc.subcore_barrier()

        def load_gather(j, b):
            pltpu.sync_copy(src_hbm.at[base + j], idx_s[b])
            pltpu.sync_copy(dst_hbm.at[base + j], idx_d.at[b])
            pltpu.async_copy(tab_hbm.at[idx_s[b]], rows.at[b], gsem).wait()

        def scatter_start(b):
            pltpu.async_copy(rows.at[b], acc.at[idx_d.at[b]], ssem[b],
                             add=True)

        def scatter_wait(b):
            pltpu.make_async_copy(tab_hbm.at[pl.ds(0, BATCH)], rows.at[b],
                                  ssem[b]).wait()

        for b in range(2):
            load_gather(b, b)
            scatter_start(b)

        def body(g, carry):
            for b in range(2):
                j = 2 * g + b
                scatter_wait(b)
                load_gather(j, b)
                scatter_start(b)
            return carry

        lax.fori_loop(1, (nb - 1) // 2, body, 0)

        scatter_wait(0)
        load_gather(nb - 1, 0)
        scatter_start(0)
        scatter_wait(0)
        scatter_wait(1)
        plsc.subcore_barrier()

        pltpu.sync_copy(
            acc.at[pl.ds(s * ROWS_PER_TILE, ROWS_PER_TILE)],
            out_hbm.at[c, pl.ds(s * ROWS_PER_TILE, ROWS_PER_TILE)],
        )

    return k(h_aug, src2d, dst2d, zinit)


def _tc_finish_body(p0_ref, p1_ref, h_ref, w_ref, o_ref):
    p = p0_ref[...] + p1_ref[...]
    ssum = p[:, :IN_FEAT]
    deg = p[:, IN_FEAT:IN_FEAT + 1]
    r = jnp.maximum(deg, 1.0)
    agg = jnp.dot(ssum / r, w_ref[...], preferred_element_type=jnp.float32)
    o_ref[...] = jnp.maximum(agg, 0.0) + h_ref[...]


def _tc_finish(p0, p1, h, W):
    blk = 1000
    grid = (N_NODES // blk,)
    return pl.pallas_call(
        _tc_finish_body,
        grid=grid,
        in_specs=[
            pl.BlockSpec((blk, D_AUG), lambda i: (i, 0)),
            pl.BlockSpec((blk, D_AUG), lambda i: (i, 0)),
            pl.BlockSpec((blk, IN_FEAT), lambda i: (i, 0)),
            pl.BlockSpec((IN_FEAT, OUT2), lambda i: (0, 0)),
        ],
        out_specs=pl.BlockSpec((blk, OUT2), lambda i: (i, 0)),
        out_shape=jax.ShapeDtypeStruct((N_NODES, OUT2), jnp.float32),
    )(p0, p1, h, W)


@jax.jit
def kernel(h, edge_index, W):
    ei = edge_index.astype(jnp.int32)
    src = ei[0]
    dst = ei[1]
    pad = E_PAD - N_EDGES
    src_p = jnp.concatenate([src, jnp.full((pad,), N_NODES, jnp.int32)])
    dst_p = jnp.concatenate([dst, jnp.zeros((pad,), jnp.int32)])
    src2d = src_p.reshape(-1, BATCH)
    dst2d = dst_p.reshape(-1, BATCH)

    h_aug = jnp.zeros((N_TAB, D_AUG), jnp.float32)
    h_aug = h_aug.at[:N_NODES, :IN_FEAT].set(h)
    h_aug = h_aug.at[:N_NODES, IN_FEAT].set(1.0)

    zinit = jnp.zeros((ROWS_PER_TILE, D_AUG), jnp.float32)

    partials = _sc_scatter(h_aug, src2d, dst2d, zinit)
    return _tc_finish(partials[0], partials[1], h, W)


# speedup vs baseline: 1.1920x; 1.0307x over previous
import functools

import jax
import jax.numpy as jnp
from jax import lax
from jax.experimental import pallas as pl
from jax.experimental.pallas import tpu as pltpu
from jax.experimental.pallas import tpu_sc as plsc

N_NODES = 10000
N_EDGES = 320000
IN_FEAT = 128
OUT2 = 128

D_AUG = 144
N_TAB = N_NODES + 16
ACC_ROWS = 10240
ROWS_PER_TILE = ACC_ROWS // 16
BATCH = 128

NC, NS = 2, 16
NW = NC * NS
BATCHES_PER_SUBCORE = 158
NB0, NB1 = 103, 55
E_PAD = NS * BATCHES_PER_SUBCORE * BATCH


def _sc_scatter(h_aug, src2d, dst2d, zinit):
    mesh = plsc.VectorSubcoreMesh(core_axis_name="c", subcore_axis_name="s")

    @functools.partial(
        pl.kernel,
        mesh=mesh,
        out_type=jax.ShapeDtypeStruct((NC, ACC_ROWS, D_AUG), jnp.float32),
        scratch_types=[
            pltpu.VMEM((BATCH,), jnp.int32),
            pltpu.VMEM((BATCH,), jnp.int32),
            pltpu.VMEM((2, BATCH), jnp.int32),
            pltpu.VMEM((2, BATCH, D_AUG), jnp.float32),
            pltpu.VMEM_SHARED((ACC_ROWS, D_AUG), jnp.float32),
            pltpu.SemaphoreType.DMA,
            [pltpu.SemaphoreType.DMA] * 2,
        ],
        compiler_params=pltpu.CompilerParams(use_tc_tiling_on_sc=False),
    )
    def k(tab_hbm, src_hbm, dst_hbm, z_hbm, out_hbm, idx_s0, idx_s1, idx_d,
          rows, acc, gsem, ssem):
        c = lax.axis_index("c")
        s = lax.axis_index("s")
        base = s * BATCHES_PER_SUBCORE + c * NB0
        nb = jnp.where(c == 0, NB0, NB1)
        idx_s = (idx_s0, idx_s1)

        pltpu.sync_copy(z_hbm, acc.at[pl.ds(s * ROWS_PER_TILE, ROWS_PER_TILE)])
        plsc.subcore_barrier()

        def load_gather(j, b):
            pltpu.sync_copy(src_hbm.at[base + j], idx_s[b])
            pltpu.sync_copy(dst_hbm.at[base + j], idx_d.at[b])
            pltpu.async_copy(tab_hbm.at[idx_s[b]], rows.at[b], gsem).wait()

        def scatter_start(b):
            pltpu.async_copy(rows.at[b], acc.at[idx_d.at[b]], ssem[b],
                             add=True)

        def scatter_wait(b):
            pltpu.make_async_copy(tab_hbm.at[pl.ds(0, BATCH)], rows.at[b],
                                  ssem[b]).wait()

        for b in range(2):
            load_gather(b, b)
            scatter_start(b)

        def body(g, carry):
            for b in range(2):
                j = 2 * g + b
                scatter_wait(b)
                load_gather(j, b)
                scatter_start(b)
            return carry

        lax.fori_loop(1, (nb - 1) // 2, body, 0)

        scatter_wait(0)
        load_gather(nb - 1, 0)
        scatter_start(0)
        scatter_wait(0)
        scatter_wait(1)
        plsc.subcore_barrier()

        pltpu.sync_copy(
            acc.at[pl.ds(s * ROWS_PER_TILE, ROWS_PER_TILE)],
            out_hbm.at[c, pl.ds(s * ROWS_PER_TILE, ROWS_PER_TILE)],
        )

    return k(h_aug, src2d, dst2d, zinit)


def _tc_finish_body(p0_ref, p1_ref, h_ref, w_ref, o_ref):
    p = p0_ref[...] + p1_ref[...]
    ssum = p[:, :IN_FEAT]
    deg = p[:, IN_FEAT:IN_FEAT + 1]
    r = jnp.maximum(deg, 1.0)
    agg = jnp.dot(ssum / r, w_ref[...], preferred_element_type=jnp.float32)
    o_ref[...] = jnp.maximum(agg, 0.0) + h_ref[...]


def _tc_finish(p0, p1, h, W):
    blk = 1000
    grid = (N_NODES // blk,)
    return pl.pallas_call(
        _tc_finish_body,
        grid=grid,
        in_specs=[
            pl.BlockSpec((blk, D_AUG), lambda i: (i, 0)),
            pl.BlockSpec((blk, D_AUG), lambda i: (i, 0)),
            pl.BlockSpec((blk, IN_FEAT), lambda i: (i, 0)),
            pl.BlockSpec((IN_FEAT, OUT2), lambda i: (0, 0)),
        ],
        out_specs=pl.BlockSpec((blk, OUT2), lambda i: (i, 0)),
        out_shape=jax.ShapeDtypeStruct((N_NODES, OUT2), jnp.float32),
    )(p0, p1, h, W)


@jax.jit
def kernel(h, edge_index, W):
    ei = edge_index.astype(jnp.int32)
    src = ei[0]
    dst = ei[1]
    pad = E_PAD - N_EDGES
    src_p = jnp.concatenate([src, jnp.full((pad,), N_NODES, jnp.int32)])
    dst_p = jnp.concatenate([dst, jnp.zeros((pad,), jnp.int32)])
    src2d = src_p.reshape(-1, BATCH)
    dst2d = dst_p.reshape(-1, BATCH)

    h_aug = jnp.zeros((N_TAB, D_AUG), jnp.float32)
    h_aug = h_aug.at[:N_NODES, :IN_FEAT].set(h)
    h_aug = h_aug.at[:N_NODES, IN_FEAT].set(1.0)

    zinit = jnp.zeros((ROWS_PER_TILE, D_AUG), jnp.float32)

    partials = _sc_scatter(h_aug, src2d, dst2d, zinit)
    return _tc_finish(partials[0], partials[1], h, W)
